# Initial kernel scaffold; baseline (speedup 1.0000x reference)
import jax
import jax.numpy as jnp
from jax.experimental import pallas as pl


def kernel(x, target, neg_samples, U):
    raise NotImplementedError("write your pallas kernel here")


# speedup vs baseline: 5.2412x; 5.2412x over previous
import jax
import jax.numpy as jnp
from jax.experimental import pallas as pl
from jax.experimental.pallas import tpu as pltpu

_VOC = 1000
_EMB = 64
_C = 4
_K = 20
_TB = 512


def _cbow_tile(x_ref, t_ref, neg_ref, u_ref, out_ref):
    i = pl.program_id(0)
    x = x_ref[...]
    t = t_ref[...]
    neg = neg_ref[...]
    U = u_ref[...]

    lane = jax.lax.broadcasted_iota(jnp.int32, (_TB, _VOC), 1)

    cnt_x = jnp.zeros((_TB, _VOC), jnp.float32)
    for c in range(_C):
        cnt_x += (x[:, c:c + 1] == lane).astype(jnp.float32)

    h = jax.lax.dot_general(
        cnt_x, U, (((1,), (0,)), ((), ())),
        preferred_element_type=jnp.float32,
        precision=jax.lax.Precision.HIGHEST) * (1.0 / _C)
    s = jax.lax.dot_general(
        h, U, (((1,), (1,)), ((), ())),
        preferred_element_type=jnp.float32,
        precision=jax.lax.Precision.HIGHEST)

    onehot_t = (t == lane).astype(jnp.float32)
    cnt_n = jnp.zeros((_TB, _VOC), jnp.float32)
    for k in range(_K):
        cnt_n += (neg[:, k:k + 1] == lane).astype(jnp.float32)

    sp = jnp.maximum(s, 0.0) + jnp.log1p(jnp.exp(-jnp.abs(s)))
    contrib = onehot_t * s - (onehot_t + cnt_n) * sp
    tile_sum = jnp.sum(contrib)

    @pl.when(i == 0)
    def _():
        out_ref[0, 0] = 0.0

    out_ref[0, 0] -= tile_sum


def kernel(x, target, neg_samples, U):
    B = x.shape[0]
    grid = (B // _TB,)
    out = pl.pallas_call(
        _cbow_tile,
        grid=grid,
        in_specs=[
            pl.BlockSpec((_TB, _C), lambda i: (i, 0)),
            pl.BlockSpec((_TB, 1), lambda i: (i, 0)),
            pl.BlockSpec((_TB, _K), lambda i: (i, 0)),
            pl.BlockSpec((_VOC, _EMB), lambda i: (0, 0)),
        ],
        out_specs=pl.BlockSpec(memory_space=pltpu.SMEM),
        out_shape=jax.ShapeDtypeStruct((1, 1), jnp.float32),
    )(x, target.reshape(B, 1), neg_samples, U)
    return out[0, 0]
